# Initial kernel scaffold; baseline (speedup 1.0000x reference)
import jax
import jax.numpy as jnp
from jax.experimental import pallas as pl


def kernel(x, edge_index, W1, b1, W2, b2, gamma, beta):
    raise NotImplementedError("write your pallas kernel here")


# speedup vs baseline: 16.1543x; 16.1543x over previous
import functools

import jax
import jax.numpy as jnp
from jax import lax
from jax.experimental import pallas as pl
from jax.experimental.pallas import tpu as pltpu
from jax.experimental.pallas import tpu_sc as plsc

N = 10000
E = 320000
D = 128
NC = 2
NS = 16
NW = NC * NS
NPAD = 10240
RPT = NPAD // NS
CHUNK = 128
EPT = 10368
NCHUNKS = EPT // CHUNK
EPAD = NW * EPT
RB = 1024
GRID = NPAD // RB

_mesh = plsc.VectorSubcoreMesh(core_axis_name="c", subcore_axis_name="s")


@functools.partial(
    pl.kernel,
    out_type=jax.ShapeDtypeStruct((NC, NPAD), jnp.float32),
    mesh=_mesh,
    scratch_types=[
        pltpu.VMEM((NCHUNKS, CHUNK), jnp.int32),
        pltpu.VMEM((CHUNK,), jnp.float32),
        pltpu.VMEM_SHARED((NPAD,), jnp.float32),
    ],
)
def _deg(dst_hbm, zeros_hbm, out_hbm, dst_v, ones_v, acc):
    c = lax.axis_index("c")
    s = lax.axis_index("s")
    wid = s * NC + c
    pltpu.sync_copy(zeros_hbm, acc.at[pl.ds(s * RPT, RPT)])
    pltpu.sync_copy(dst_hbm.at[wid], dst_v)
    for i in range(CHUNK // 16):
        ones_v[pl.ds(i * 16, 16)] = jnp.ones((16,), jnp.float32)
    plsc.subcore_barrier()

    def body(j, _):
        pltpu.sync_copy(ones_v, acc.at[dst_v.at[j]], add=True)
        return ()

    lax.fori_loop(0, NCHUNKS, body, ())
    plsc.subcore_barrier()
    pltpu.sync_copy(acc.at[pl.ds(s * RPT, RPT)],
                    out_hbm.at[c, pl.ds(s * RPT, RPT)])


@functools.partial(
    pl.kernel,
    out_type=jax.ShapeDtypeStruct((NC, NPAD, D), jnp.float32),
    mesh=_mesh,
    scratch_types=[
        pltpu.VMEM((NCHUNKS, CHUNK), jnp.int32),
        pltpu.VMEM((NCHUNKS, CHUNK), jnp.int32),
        pltpu.VMEM((CHUNK, D), jnp.float32),
        pltpu.VMEM_SHARED((NPAD, D), jnp.float32),
        pltpu.SemaphoreType.DMA,
    ],
)
def _agg(src_hbm, dst_hbm, h_hbm, zeros_hbm, out_hbm,
         src_v, dst_v, rows_v, acc, sem):
    c = lax.axis_index("c")
    s = lax.axis_index("s")
    wid = s * NC + c
    pltpu.sync_copy(zeros_hbm, acc.at[pl.ds(s * RPT, RPT)])
    pltpu.sync_copy(src_hbm.at[wid], src_v)
    pltpu.sync_copy(dst_hbm.at[wid], dst_v)
    plsc.subcore_barrier()

    def body(j, _):
        pltpu.async_copy(h_hbm.at[src_v.at[j]], rows_v, sem).wait()
        pltpu.sync_copy(rows_v, acc.at[dst_v.at[j]], add=True)
        return ()

    lax.fori_loop(0, NCHUNKS, body, ())
    plsc.subcore_barrier()
    pltpu.sync_copy(acc.at[pl.ds(s * RPT, RPT)],
                    out_hbm.at[c, pl.ds(s * RPT, RPT)])


def _dinv_body(degp_ref, dinv_ref):
    dp = degp_ref[...]
    d = dp[:NPAD] + dp[NPAD:]
    dinv_ref[...] = jnp.where(d > 0.0, lax.rsqrt(jnp.maximum(d, 1e-30)), 0.0)


def _mm_scale_body(x_ref, w_ref, dinv_ref, o_ref):
    h = jnp.dot(x_ref[...], w_ref[...], preferred_element_type=jnp.float32)
    o_ref[...] = h * dinv_ref[...]


def _combine_stats_body(ap_ref, dinv_ref, b_ref, o_ref, s1_ref, s2_ref):
    i = pl.program_id(0)
    ap = ap_ref[...]
    o = (ap[0] + ap[1]) * dinv_ref[...] + b_ref[...]
    o_ref[...] = o
    row = i * RB + lax.broadcasted_iota(jnp.int32, (RB, D), 0)
    om = jnp.where(row < N, o, 0.0)

    @pl.when(i == 0)
    def _():
        s1_ref[...] = jnp.zeros_like(s1_ref)
        s2_ref[...] = jnp.zeros_like(s2_ref)

    s1_ref[...] += jnp.sum(om, axis=0, keepdims=True)
    s2_ref[...] += jnp.sum(om * om, axis=0, keepdims=True)


def _bn_mm_body(o_ref, s1_ref, s2_ref, g_ref, be_ref, w_ref, dinv_ref, h_ref):
    mean = s1_ref[...] / N
    var = s2_ref[...] / N - mean * mean
    rstd = lax.rsqrt(var + 1e-5)
    y = (o_ref[...] - mean) * (rstd * g_ref[...]) + be_ref[...]
    y = jnp.maximum(y, 0.0)
    h = jnp.dot(y, w_ref[...], preferred_element_type=jnp.float32)
    h_ref[...] = h * dinv_ref[...]


def _final_body(ap_ref, dinv_ref, b_ref, o_ref):
    ap = ap_ref[...]
    o_ref[...] = (ap[0] + ap[1]) * dinv_ref[...] + b_ref[...]


def kernel(x, edge_index, W1, b1, W2, b2, gamma, beta):
    f32 = jnp.float32
    src = edge_index[0].astype(jnp.int32)
    dst = edge_index[1].astype(jnp.int32)
    loops = jnp.arange(N, dtype=jnp.int32)
    npad_e = EPAD - E - N
    src_a = jnp.concatenate(
        [src, loops, jnp.zeros((npad_e,), jnp.int32)]).reshape(NW, NCHUNKS, CHUNK)
    dst_a = jnp.concatenate(
        [dst, loops, jnp.full((npad_e,), N, jnp.int32)]).reshape(NW, NCHUNKS, CHUNK)
    x_p = jnp.pad(x, ((0, NPAD - N), (0, 0)))
    z1 = jnp.zeros((RPT,), f32)
    z2 = jnp.zeros((RPT, D), f32)
    b1r = b1.reshape(1, D)
    b2r = b2.reshape(1, D)
    gr = gamma.reshape(1, D)
    ber = beta.reshape(1, D)

    deg_p = _deg(dst_a, z1)

    dinv = pl.pallas_call(
        _dinv_body,
        out_shape=jax.ShapeDtypeStruct((NPAD, 1), f32),
    )(deg_p.reshape(NC * NPAD, 1))

    row_spec = pl.BlockSpec((RB, D), lambda i: (i, 0))
    vec_spec = pl.BlockSpec((RB, 1), lambda i: (i, 0))
    full_spec = pl.BlockSpec((1, D), lambda i: (0, 0))
    w_spec = pl.BlockSpec((D, D), lambda i: (0, 0))
    part_spec = pl.BlockSpec((NC, RB, D), lambda i: (0, i, 0))

    h1 = pl.pallas_call(
        _mm_scale_body,
        grid=(GRID,),
        in_specs=[row_spec, w_spec, vec_spec],
        out_specs=row_spec,
        out_shape=jax.ShapeDtypeStruct((NPAD, D), f32),
    )(x_p, W1, dinv)

    agg1 = _agg(src_a, dst_a, h1, z2)

    out1, s1, s2 = pl.pallas_call(
        _combine_stats_body,
        grid=(GRID,),
        in_specs=[part_spec, vec_spec, full_spec],
        out_specs=[row_spec, full_spec, full_spec],
        out_shape=[
            jax.ShapeDtypeStruct((NPAD, D), f32),
            jax.ShapeDtypeStruct((1, D), f32),
            jax.ShapeDtypeStruct((1, D), f32),
        ],
        compiler_params=pltpu.CompilerParams(
            dimension_semantics=("arbitrary",)),
    )(agg1, dinv, b1r)

    h2 = pl.pallas_call(
        _bn_mm_body,
        grid=(GRID,),
        in_specs=[row_spec, full_spec, full_spec, full_spec, full_spec,
                  w_spec, vec_spec],
        out_specs=row_spec,
        out_shape=jax.ShapeDtypeStruct((NPAD, D), f32),
    )(out1, s1, s2, gr, ber, W2, dinv)

    agg2 = _agg(src_a, dst_a, h2, z2)

    out_p = pl.pallas_call(
        _final_body,
        grid=(GRID,),
        in_specs=[part_spec, vec_spec, full_spec],
        out_specs=row_spec,
        out_shape=jax.ShapeDtypeStruct((NPAD, D), f32),
    )(agg2, dinv, b2r)

    return out_p[:N]
